# baseline (speedup 1.0000x reference)
import jax
import jax.numpy as jnp
from jax.experimental import pallas as pl
from jax.experimental.pallas import tpu as pltpu

_N_BOX = 20000
_N_PAD = 20480
_ROWS = 160
_NCLS = 20
_K = 1024
_PRE_NMS = 1000
_MAX_DET = 300
_OUT_SLOTS = 384
_NMS_T = 0.3
_SCORE_T = 0.05
_KEY_LO = 0x3D4CCCCD
_KEY_HI = 0x3F800001


def _iota(shape, dim):
    return jax.lax.broadcasted_iota(jnp.int32, shape, dim)


def _sel_body(p_ref, li_ref):
    f32 = jnp.float32
    P = p_ref[...]
    valid = P > _SCORE_T
    key = jnp.where(valid, jax.lax.bitcast_convert_type(P, jnp.int32),
                    jnp.int32(-1))
    nt = jnp.sum(valid.astype(jnp.int32), axis=(1, 2), keepdims=True)
    kq = jnp.minimum(jnp.int32(_PRE_NMS), nt)

    def bs_body(_, lh):
        lo, hi = lh
        mid = lo + (hi - lo + 1) // 2
        cnt = jnp.sum((key >= mid).astype(jnp.int32), axis=(1, 2),
                      keepdims=True)
        good = cnt >= kq
        return (jnp.where(good, mid, lo), jnp.where(good, hi, mid - 1))

    V, _ = jax.lax.fori_loop(
        0, 26, bs_body,
        (jnp.full((_NCLS, 1, 1), _KEY_LO, jnp.int32),
         jnp.full((_NCLS, 1, 1), _KEY_HI, jnp.int32)))

    gt = key > V
    n1 = jnp.sum(gt.astype(jnp.int32), axis=(1, 2), keepdims=True)
    eqm = key == V

    TRIU = (_iota((128, 128), 0) <= _iota((128, 128), 1)).astype(f32)
    UTS = (_iota((_ROWS, _ROWS), 0) < _iota((_ROWS, _ROWS), 1)).astype(f32)

    def cum(m3):
        m2 = m3.reshape(_NCLS * _ROWS, 128)
        li2 = jax.lax.dot_general(m2, TRIU, (((1,), (0,)), ((), ())),
                                  preferred_element_type=f32)
        rt = li2[:, 127:128].reshape(_NCLS, _ROWS)
        rp = jax.lax.dot_general(rt, UTS, (((1,), (0,)), ((), ())),
                                 preferred_element_type=f32)
        return li2.reshape(_NCLS, _ROWS, 128), rp.reshape(_NCLS, _ROWS, 1)

    eqf = eqm.astype(f32)
    li_eq, rp_eq = cum(eqf)
    ex_eq = li_eq + rp_eq - eqf
    rem = (kq - n1).astype(f32)
    sel = gt | (eqm & (ex_eq < rem))
    li3, _ = cum(sel.astype(f32))
    li_ref[...] = li3


def _nms_body(li_ref, p_ref, y1_ref, x1_ref, y2_ref, x2_ref, out_ref):
    f32 = jnp.float32
    li = li_ref[0]
    p = p_ref[0]
    y1 = y1_ref[0]
    x1 = x1_ref[0]
    y2 = y2_ref[0]
    x2 = x2_ref[0]

    cnt_r = li[:, 127:128]
    LTS = (_iota((_ROWS, _ROWS), 0) > _iota((_ROWS, _ROWS), 1)).astype(f32)
    rp = jax.lax.dot_general(LTS, cnt_r, (((1,), (0,)), ((), ())),
                             preferred_element_type=f32)

    tgt = (_iota((_ROWS, 128), 1) + 1).astype(f32)
    lo_l = jnp.zeros((_ROWS, 128), jnp.int32)
    hi_l = jnp.full((_ROWS, 128), 127, jnp.int32)
    for _ in range(7):
        mid = (lo_l + hi_l) // 2
        val = jnp.take_along_axis(li, mid, axis=1)
        ge = val >= tgt
        hi_l = jnp.where(ge, mid, hi_l)
        lo_l = jnp.where(ge, lo_l, mid + 1)
    src = lo_l

    ones_pl = jnp.ones((_ROWS, 128), f32)
    planes = [p, y1, x1, y2, x2, ones_pl]
    rc = [jnp.take_along_axis(z, src, axis=1) for z in planes]

    rp_i = rp.astype(jnp.int32)
    sh = rp_i % 128
    orow = rp_i // 128
    lane = _iota((_ROWS, 128), 1)
    ridx = (lane - sh + 128) % 128
    in_cnt = ridx.astype(f32) < cnt_r
    m1 = ((lane >= sh) & in_cnt).astype(f32)
    m2 = ((lane < sh) & in_cnt).astype(f32)

    orow_row = orow.astype(f32).reshape(1, _ROWS)
    io8 = _iota((8, _ROWS), 0).astype(f32)
    OHR1 = (io8 == orow_row).astype(f32)
    OHR2 = (io8 == orow_row + 1.0).astype(f32)

    def dispatch(z):
        rolled = jnp.take_along_axis(z, ridx, axis=1)
        a = jax.lax.dot_general(OHR1, rolled * m1, (((1,), (0,)), ((), ())),
                                precision=jax.lax.Precision.HIGHEST,
                                preferred_element_type=f32)
        b = jax.lax.dot_general(OHR2, rolled * m2, (((1,), (0,)), ((), ())),
                                precision=jax.lax.Precision.HIGHEST,
                                preferred_element_type=f32)
        return a + b

    comp = [dispatch(z) for z in rc]

    crow = [z.reshape(1, _K) for z in comp]

    s_row = crow[0]
    cols8 = [jnp.transpose(comp[0][i:i + 1], (1, 0)) for i in range(8)]
    s_col = jnp.concatenate(cols8, axis=0)
    ij = _iota((_K, _K), 0)
    ik = _iota((_K, _K), 1)
    ij_lt_ik = ij < ik
    G = ((s_col > s_row) | ((s_col == s_row) & ij_lt_ik)).astype(f32)
    rank = jnp.sum(G, axis=0, keepdims=True)
    OHS = (ij.astype(f32) == rank).astype(f32)
    P8 = jnp.concatenate(crow + [jnp.zeros((2, _K), f32)], axis=0)
    S = jax.lax.dot_general(OHS, P8, (((1,), (1,)), ((), ())),
                            precision=jax.lax.Precision.HIGHEST,
                            preferred_element_type=f32)

    sc = S[:, 0:1]
    sy1 = S[:, 1:2]
    sx1 = S[:, 2:3]
    sy2 = S[:, 3:4]
    sx2 = S[:, 4:5]
    s_sorted = sc.reshape(1, _K)
    y1r = sy1.reshape(1, _K)
    x1r = sx1.reshape(1, _K)
    y2r = sy2.reshape(1, _K)
    x2r = sx2.reshape(1, _K)

    ih = jnp.minimum(sy2, y2r) - jnp.maximum(sy1, y1r)
    iw = jnp.minimum(sx2, x2r) - jnp.maximum(sx1, x1r)
    inter = jnp.maximum(ih, 0.0) * jnp.maximum(iw, 0.0)
    area_c = (sy2 - sy1) * (sx2 - sx1)
    area_r = (y2r - y1r) * (x2r - x1r)
    iou = inter / (area_c + area_r - inter + 1e-9)
    SUPB = iou > _NMS_T

    UT128 = _iota((128, 128), 0) < _iota((128, 128), 1)
    K0 = s_sorted > 0.0
    K = K0
    for b in range(8):
        lo, hi = 128 * b, 128 * (b + 1)
        Db = SUPB[lo:hi, lo:hi] & UT128
        kb0 = K[:, lo:hi]

        def step(kf):
            kcol = jnp.transpose(kf, (1, 0)) > 0.5
            sup = jnp.any(Db & kcol, axis=0, keepdims=True)
            return (kb0 & jnp.logical_not(sup)).astype(f32)

        kb0f = kb0.astype(f32)
        k1 = step(kb0f)
        k2 = step(k1)

        def in_cond(c):
            kp, k, it = c
            return jnp.logical_and(it < 130, jnp.any(k != kp))

        def in_body(c):
            _, k, it = c
            ka = step(k)
            return (ka, step(ka), it + 2)

        _, kbf, _ = jax.lax.while_loop(in_cond, in_body,
                                       (k1, k2, jnp.int32(2)))
        kb = kbf > 0.5

        if b < 7:
            kbcol = jnp.transpose(kb.astype(f32), (1, 0)) > 0.5
            supL = jnp.any(SUPB[lo:hi, hi:] & kbcol, axis=0, keepdims=True)
            later = K[:, hi:] & jnp.logical_not(supL)
            if b == 0:
                K = jnp.concatenate([kb, later], axis=1)
            else:
                K = jnp.concatenate([K[:, :lo], kb, later], axis=1)
        else:
            K = jnp.concatenate([K[:, :lo], kb], axis=1)

    Kf = K.astype(f32)
    TRIUK = (ij <= ik).astype(f32)
    inclK = jax.lax.dot_general(Kf, TRIUK, (((1,), (0,)), ((), ())),
                                preferred_element_type=f32)
    pos2 = jnp.where(K, inclK - 1.0, -1.0)
    OH2 = (_iota((_OUT_SLOTS, _K), 0).astype(f32) == pos2).astype(f32)
    OUT = jax.lax.dot_general(OH2, S, (((1,), (0,)), ((), ())),
                              precision=jax.lax.Precision.HIGHEST,
                              preferred_element_type=f32)
    out_ref[0] = OUT


def kernel(raw_cls_bbox, raw_prob):
    probs_t = raw_prob.T[1:]
    P = jnp.pad(probs_t, ((0, 0), (0, _N_PAD - _N_BOX)))
    P = P.reshape(_NCLS, _ROWS, 128)
    bb = jnp.transpose(raw_cls_bbox, (1, 2, 0))[1:]
    bb = jnp.pad(bb, ((0, 0), (0, 0), (0, _N_PAD - _N_BOX)))
    bb = bb.reshape(_NCLS, 4, _ROWS, 128)

    li = pl.pallas_call(
        _sel_body,
        out_shape=jax.ShapeDtypeStruct((_NCLS, _ROWS, 128), jnp.float32),
    )(P)

    spec = pl.BlockSpec((1, _ROWS, 128), lambda i: (i, 0, 0))
    out = pl.pallas_call(
        _nms_body,
        out_shape=jax.ShapeDtypeStruct((_NCLS, _OUT_SLOTS, 8), jnp.float32),
        grid=(_NCLS,),
        in_specs=[spec] * 6,
        out_specs=pl.BlockSpec((1, _OUT_SLOTS, 8), lambda i: (i, 0, 0)),
    )(li, P, bb[:, 0], bb[:, 1], bb[:, 2], bb[:, 3])

    return jnp.concatenate(
        [out[:, :_MAX_DET, 1:5], out[:, :_MAX_DET, 0:1]], axis=-1)


# speedup vs baseline: 1.1981x; 1.1981x over previous
import jax
import jax.numpy as jnp
from jax.experimental import pallas as pl
from jax.experimental.pallas import tpu as pltpu

_N_BOX = 20000
_N_PAD = 20480
_ROWS = 160
_NCLS = 20
_K = 1024
_PRE_NMS = 1000
_MAX_DET = 300
_OUT_SLOTS = 384
_NMS_T = 0.3
_SCORE_T = 0.05
_KEY_LO = 0x3D4CCCCD
_KEY_HI = 0x3F800001


def _iota(shape, dim):
    return jax.lax.broadcasted_iota(jnp.int32, shape, dim)


def _split3(x):
    f32 = jnp.float32
    hi = x.astype(jnp.bfloat16).astype(f32)
    r = x - hi
    mid = r.astype(jnp.bfloat16).astype(f32)
    lo = r - mid
    return hi, mid, lo


def _onehot_matmul_nt(oh, payload):
    f32 = jnp.float32
    hi, mid, lo = _split3(payload)
    dn = (((1,), (1,)), ((), ()))
    a = jax.lax.dot_general(oh, hi, dn, preferred_element_type=f32)
    b = jax.lax.dot_general(oh, mid, dn, preferred_element_type=f32)
    c = jax.lax.dot_general(oh, lo, dn, preferred_element_type=f32)
    return a + (b + c)


def _onehot_matmul(oh, payload):
    f32 = jnp.float32
    hi, mid, lo = _split3(payload)
    dn = (((1,), (0,)), ((), ()))
    a = jax.lax.dot_general(oh, hi, dn, preferred_element_type=f32)
    b = jax.lax.dot_general(oh, mid, dn, preferred_element_type=f32)
    c = jax.lax.dot_general(oh, lo, dn, preferred_element_type=f32)
    return a + (b + c)


def _sel_body(p_ref, li_ref):
    f32 = jnp.float32
    P = p_ref[...]
    valid = P > _SCORE_T
    key = jnp.where(valid, jax.lax.bitcast_convert_type(P, jnp.int32),
                    jnp.int32(-1))
    nt = jnp.sum(valid.astype(jnp.int32), axis=(1, 2), keepdims=True)
    kq = jnp.minimum(jnp.int32(_PRE_NMS), nt)

    def bs_body(_, lh):
        lo, hi = lh
        mid = lo + (hi - lo + 1) // 2
        cnt = jnp.sum((key >= mid).astype(jnp.int32), axis=(1, 2),
                      keepdims=True)
        good = cnt >= kq
        return (jnp.where(good, mid, lo), jnp.where(good, hi, mid - 1))

    V, _ = jax.lax.fori_loop(
        0, 26, bs_body,
        (jnp.full((_NCLS, 1, 1), _KEY_LO, jnp.int32),
         jnp.full((_NCLS, 1, 1), _KEY_HI, jnp.int32)))

    gt = key > V
    n1 = jnp.sum(gt.astype(jnp.int32), axis=(1, 2), keepdims=True)
    eqm = key == V

    TRIU = (_iota((128, 128), 0) <= _iota((128, 128), 1)).astype(f32)
    UTS = (_iota((_ROWS, _ROWS), 0) < _iota((_ROWS, _ROWS), 1)).astype(f32)

    def cum(m3):
        m2 = m3.reshape(_NCLS * _ROWS, 128)
        li2 = jax.lax.dot_general(m2, TRIU, (((1,), (0,)), ((), ())),
                                  preferred_element_type=f32)
        rt = li2[:, 127:128].reshape(_NCLS, _ROWS)
        rp = jax.lax.dot_general(rt, UTS, (((1,), (0,)), ((), ())),
                                 preferred_element_type=f32)
        return li2.reshape(_NCLS, _ROWS, 128), rp.reshape(_NCLS, _ROWS, 1)

    eqf = eqm.astype(f32)
    li_eq, rp_eq = cum(eqf)
    ex_eq = li_eq + rp_eq - eqf
    rem = (kq - n1).astype(f32)
    sel = gt | (eqm & (ex_eq < rem))
    li3, _ = cum(sel.astype(f32))
    li_ref[...] = li3


def _nms_body(li_ref, p_ref, y1_ref, x1_ref, y2_ref, x2_ref, out_ref):
    f32 = jnp.float32
    li = li_ref[0]
    p = p_ref[0]
    y1 = y1_ref[0]
    x1 = x1_ref[0]
    y2 = y2_ref[0]
    x2 = x2_ref[0]

    cnt_r = li[:, 127:128]
    LTS = (_iota((_ROWS, _ROWS), 0) > _iota((_ROWS, _ROWS), 1)).astype(f32)
    rp = jax.lax.dot_general(LTS, cnt_r, (((1,), (0,)), ((), ())),
                             preferred_element_type=f32)

    tgt = (_iota((_ROWS, 128), 1) + 1).astype(f32)
    lo_l = jnp.zeros((_ROWS, 128), jnp.int32)
    hi_l = jnp.full((_ROWS, 128), 127, jnp.int32)
    for _ in range(7):
        mid = (lo_l + hi_l) // 2
        val = jnp.take_along_axis(li, mid, axis=1)
        ge = val >= tgt
        hi_l = jnp.where(ge, mid, hi_l)
        lo_l = jnp.where(ge, lo_l, mid + 1)
    src = lo_l

    ones_pl = jnp.ones((_ROWS, 128), f32)
    planes = [p, y1, x1, y2, x2, ones_pl]
    rc = [jnp.take_along_axis(z, src, axis=1) for z in planes]

    rp_i = rp.astype(jnp.int32)
    sh = rp_i % 128
    orow = rp_i // 128
    lane = _iota((_ROWS, 128), 1)
    ridx = (lane - sh + 128) % 128
    in_cnt = ridx.astype(f32) < cnt_r
    m1 = ((lane >= sh) & in_cnt).astype(f32)
    m2 = ((lane < sh) & in_cnt).astype(f32)

    orow_row = orow.astype(f32).reshape(1, _ROWS)
    io8 = _iota((8, _ROWS), 0).astype(f32)
    OHR1 = (io8 == orow_row).astype(f32)
    OHR2 = (io8 == orow_row + 1.0).astype(f32)

    def dispatch(z):
        rolled = jnp.take_along_axis(z, ridx, axis=1)
        a = jax.lax.dot_general(OHR1, rolled * m1, (((1,), (0,)), ((), ())),
                                precision=jax.lax.Precision.HIGHEST,
                                preferred_element_type=f32)
        b = jax.lax.dot_general(OHR2, rolled * m2, (((1,), (0,)), ((), ())),
                                precision=jax.lax.Precision.HIGHEST,
                                preferred_element_type=f32)
        return a + b

    comp = [dispatch(z) for z in rc]

    crow = [z.reshape(1, _K) for z in comp]

    s_row = crow[0]
    cols8 = [jnp.transpose(comp[0][i:i + 1], (1, 0)) for i in range(8)]
    s_col = jnp.concatenate(cols8, axis=0)
    ij = _iota((_K, _K), 0)
    ik = _iota((_K, _K), 1)
    ij_lt_ik = ij < ik
    G = ((s_col > s_row) | ((s_col == s_row) & ij_lt_ik)).astype(f32)
    rank = jnp.sum(G, axis=0, keepdims=True)
    OHS = (ij.astype(f32) == rank).astype(f32)
    P8 = jnp.concatenate(crow + [jnp.zeros((2, _K), f32)], axis=0)
    S = _onehot_matmul_nt(OHS, P8)

    sc = S[:, 0:1]
    sy1 = S[:, 1:2]
    sx1 = S[:, 2:3]
    sy2 = S[:, 3:4]
    sx2 = S[:, 4:5]
    s_sorted = sc.reshape(1, _K)
    y1r = sy1.reshape(1, _K)
    x1r = sx1.reshape(1, _K)
    y2r = sy2.reshape(1, _K)
    x2r = sx2.reshape(1, _K)

    ih = jnp.minimum(sy2, y2r) - jnp.maximum(sy1, y1r)
    iw = jnp.minimum(sx2, x2r) - jnp.maximum(sx1, x1r)
    inter = jnp.maximum(ih, 0.0) * jnp.maximum(iw, 0.0)
    area_c = (sy2 - sy1) * (sx2 - sx1)
    area_r = (y2r - y1r) * (x2r - x1r)
    iou = inter / (area_c + area_r - inter + 1e-9)
    SUPB = iou > _NMS_T

    UT128 = _iota((128, 128), 0) < _iota((128, 128), 1)
    K0 = s_sorted > 0.0
    K = K0
    for b in range(8):
        lo, hi = 128 * b, 128 * (b + 1)
        Db = SUPB[lo:hi, lo:hi] & UT128
        kb0 = K[:, lo:hi]

        def step(kf):
            kcol = jnp.transpose(kf, (1, 0)) > 0.5
            sup = jnp.any(Db & kcol, axis=0, keepdims=True)
            return (kb0 & jnp.logical_not(sup)).astype(f32)

        kb0f = kb0.astype(f32)
        k1 = step(kb0f)
        k2 = step(k1)

        def in_cond(c):
            kp, k, it = c
            return jnp.logical_and(it < 130, jnp.any(k != kp))

        def in_body(c):
            _, k, it = c
            ka = step(k)
            return (ka, step(ka), it + 2)

        _, kbf, _ = jax.lax.while_loop(in_cond, in_body,
                                       (k1, k2, jnp.int32(2)))
        kb = kbf > 0.5

        if b < 7:
            kbcol = jnp.transpose(kb.astype(f32), (1, 0)) > 0.5
            supL = jnp.any(SUPB[lo:hi, hi:] & kbcol, axis=0, keepdims=True)
            later = K[:, hi:] & jnp.logical_not(supL)
            if b == 0:
                K = jnp.concatenate([kb, later], axis=1)
            else:
                K = jnp.concatenate([K[:, :lo], kb, later], axis=1)
        else:
            K = jnp.concatenate([K[:, :lo], kb], axis=1)

    Kf = K.astype(f32)
    TRIUK = (ij <= ik).astype(f32)
    inclK = jax.lax.dot_general(Kf, TRIUK, (((1,), (0,)), ((), ())),
                                preferred_element_type=f32)
    pos2 = jnp.where(K, inclK - 1.0, -1.0)
    OH2 = (_iota((_OUT_SLOTS, _K), 0).astype(f32) == pos2).astype(f32)
    OUT = _onehot_matmul(OH2, S)
    out_ref[0] = OUT


def kernel(raw_cls_bbox, raw_prob):
    probs_t = raw_prob.T[1:]
    P = jnp.pad(probs_t, ((0, 0), (0, _N_PAD - _N_BOX)))
    P = P.reshape(_NCLS, _ROWS, 128)
    bb = jnp.transpose(raw_cls_bbox, (1, 2, 0))[1:]
    bb = jnp.pad(bb, ((0, 0), (0, 0), (0, _N_PAD - _N_BOX)))
    bb = bb.reshape(_NCLS, 4, _ROWS, 128)

    li = pl.pallas_call(
        _sel_body,
        out_shape=jax.ShapeDtypeStruct((_NCLS, _ROWS, 128), jnp.float32),
    )(P)

    spec = pl.BlockSpec((1, _ROWS, 128), lambda i: (i, 0, 0))
    out = pl.pallas_call(
        _nms_body,
        out_shape=jax.ShapeDtypeStruct((_NCLS, _OUT_SLOTS, 8), jnp.float32),
        grid=(_NCLS,),
        in_specs=[spec] * 6,
        out_specs=pl.BlockSpec((1, _OUT_SLOTS, 8), lambda i: (i, 0, 0)),
    )(li, P, bb[:, 0], bb[:, 1], bb[:, 2], bb[:, 3])

    return jnp.concatenate(
        [out[:, :_MAX_DET, 1:5], out[:, :_MAX_DET, 0:1]], axis=-1)


# speedup vs baseline: 1.4261x; 1.1903x over previous
import jax
import jax.numpy as jnp
from jax.experimental import pallas as pl
from jax.experimental.pallas import tpu as pltpu

_N_BOX = 20000
_N_PAD = 20480
_ROWS = 160
_NCLS = 20
_K = 1024
_PRE_NMS = 1000
_MAX_DET = 300
_OUT_SLOTS = 384
_NMS_T = 0.3
_SCORE_T = 0.05
_KEY_LO = 0x3D4CCCCD
_KEY_HI = 0x3F800001


def _iota(shape, dim):
    return jax.lax.broadcasted_iota(jnp.int32, shape, dim)


def _split3(x):
    f32 = jnp.float32
    hi = x.astype(jnp.bfloat16).astype(f32)
    r = x - hi
    mid = r.astype(jnp.bfloat16).astype(f32)
    lo = r - mid
    return hi, mid, lo


def _onehot_matmul_nt(oh, payload):
    f32 = jnp.float32
    hi, mid, lo = _split3(payload)
    dn = (((1,), (1,)), ((), ()))
    a = jax.lax.dot_general(oh, hi, dn, preferred_element_type=f32)
    b = jax.lax.dot_general(oh, mid, dn, preferred_element_type=f32)
    c = jax.lax.dot_general(oh, lo, dn, preferred_element_type=f32)
    return a + (b + c)


def _onehot_matmul(oh, payload):
    f32 = jnp.float32
    hi, mid, lo = _split3(payload)
    dn = (((1,), (0,)), ((), ()))
    a = jax.lax.dot_general(oh, hi, dn, preferred_element_type=f32)
    b = jax.lax.dot_general(oh, mid, dn, preferred_element_type=f32)
    c = jax.lax.dot_general(oh, lo, dn, preferred_element_type=f32)
    return a + (b + c)


def _sel_body(p_ref, li_ref):
    f32 = jnp.float32
    P = p_ref[...]
    valid = P > _SCORE_T
    key = jnp.where(valid, jax.lax.bitcast_convert_type(P, jnp.int32),
                    jnp.int32(-1))
    nt = jnp.sum(valid.astype(jnp.int32), axis=(1, 2), keepdims=True)
    kq = jnp.minimum(jnp.int32(_PRE_NMS), nt)

    def bs_body(_, lh):
        lo, hi = lh
        mid = lo + (hi - lo + 1) // 2
        cnt = jnp.sum((key >= mid).astype(jnp.int32), axis=(1, 2),
                      keepdims=True)
        good = cnt >= kq
        return (jnp.where(good, mid, lo), jnp.where(good, hi, mid - 1))

    V, _ = jax.lax.fori_loop(
        0, 26, bs_body,
        (jnp.full((_NCLS, 1, 1), _KEY_LO, jnp.int32),
         jnp.full((_NCLS, 1, 1), _KEY_HI, jnp.int32)))

    gt = key > V
    n1 = jnp.sum(gt.astype(jnp.int32), axis=(1, 2), keepdims=True)
    eqm = key == V

    TRIU = (_iota((128, 128), 0) <= _iota((128, 128), 1)).astype(f32)
    UTS = (_iota((_ROWS, _ROWS), 0) < _iota((_ROWS, _ROWS), 1)).astype(f32)

    def cum(m3):
        m2 = m3.reshape(_NCLS * _ROWS, 128)
        li2 = jax.lax.dot_general(m2, TRIU, (((1,), (0,)), ((), ())),
                                  preferred_element_type=f32)
        rt = li2[:, 127:128].reshape(_NCLS, _ROWS)
        rp = jax.lax.dot_general(rt, UTS, (((1,), (0,)), ((), ())),
                                 preferred_element_type=f32)
        return li2.reshape(_NCLS, _ROWS, 128), rp.reshape(_NCLS, _ROWS, 1)

    eqf = eqm.astype(f32)
    li_eq, rp_eq = cum(eqf)
    ex_eq = li_eq + rp_eq - eqf
    rem = (kq - n1).astype(f32)
    sel = gt | (eqm & (ex_eq < rem))
    li3, _ = cum(sel.astype(f32))
    li_ref[...] = li3


def _nms_body(li_ref, p_ref, y1_ref, x1_ref, y2_ref, x2_ref, out_ref):
    f32 = jnp.float32
    li = li_ref[0]
    p = p_ref[0]
    y1 = y1_ref[0]
    x1 = x1_ref[0]
    y2 = y2_ref[0]
    x2 = x2_ref[0]

    cnt_r = li[:, 127:128]
    LTS = (_iota((_ROWS, _ROWS), 0) > _iota((_ROWS, _ROWS), 1)).astype(f32)
    rp = jax.lax.dot_general(LTS, cnt_r, (((1,), (0,)), ((), ())),
                             preferred_element_type=f32)

    tgt = (_iota((_ROWS, 128), 1) + 1).astype(f32)
    lo_l = jnp.zeros((_ROWS, 128), jnp.int32)
    hi_l = jnp.full((_ROWS, 128), 127, jnp.int32)
    for _ in range(7):
        mid = (lo_l + hi_l) // 2
        val = jnp.take_along_axis(li, mid, axis=1)
        ge = val >= tgt
        hi_l = jnp.where(ge, mid, hi_l)
        lo_l = jnp.where(ge, lo_l, mid + 1)
    src = lo_l

    ones_pl = jnp.ones((_ROWS, 128), f32)
    planes = [p, y1, x1, y2, x2, ones_pl]
    rc = [jnp.take_along_axis(z, src, axis=1) for z in planes]

    rp_i = rp.astype(jnp.int32)
    sh = rp_i % 128
    orow = rp_i // 128
    lane = _iota((_ROWS, 128), 1)
    ridx = (lane - sh + 128) % 128
    in_cnt = ridx.astype(f32) < cnt_r
    m1 = ((lane >= sh) & in_cnt).astype(f32)
    m2 = ((lane < sh) & in_cnt).astype(f32)

    orow_row = orow.astype(f32).reshape(1, _ROWS)
    io8 = _iota((8, _ROWS), 0).astype(f32)
    OHR1 = (io8 == orow_row).astype(f32)
    OHR2 = (io8 == orow_row + 1.0).astype(f32)

    def dispatch(z):
        rolled = jnp.take_along_axis(z, ridx, axis=1)
        a = jax.lax.dot_general(OHR1, rolled * m1, (((1,), (0,)), ((), ())),
                                precision=jax.lax.Precision.HIGHEST,
                                preferred_element_type=f32)
        b = jax.lax.dot_general(OHR2, rolled * m2, (((1,), (0,)), ((), ())),
                                precision=jax.lax.Precision.HIGHEST,
                                preferred_element_type=f32)
        return a + b

    comp = [dispatch(z) for z in rc]

    crow = [z.reshape(1, _K) for z in comp]

    s_row = crow[0]
    cols8 = [jnp.transpose(comp[0][i:i + 1], (1, 0)) for i in range(8)]
    s_col = jnp.concatenate(cols8, axis=0)
    ij = _iota((_K, _K), 0)
    ik = _iota((_K, _K), 1)
    ij_lt_ik = ij < ik
    G = ((s_col > s_row) | ((s_col == s_row) & ij_lt_ik)).astype(f32)
    rank = jnp.sum(G, axis=0, keepdims=True)
    OHS = (ij.astype(f32) == rank).astype(f32)
    P8 = jnp.concatenate(crow + [jnp.zeros((2, _K), f32)], axis=0)
    S = _onehot_matmul_nt(OHS, P8)

    sc = S[:, 0:1]
    sy1 = S[:, 1:2]
    sx1 = S[:, 2:3]
    sy2 = S[:, 3:4]
    sx2 = S[:, 4:5]
    s_sorted = sc.reshape(1, _K)
    y1r = sy1.reshape(1, _K)
    x1r = sx1.reshape(1, _K)
    y2r = sy2.reshape(1, _K)
    x2r = sx2.reshape(1, _K)

    ih = jnp.minimum(sy2, y2r) - jnp.maximum(sy1, y1r)
    iw = jnp.minimum(sx2, x2r) - jnp.maximum(sx1, x1r)
    inter = jnp.maximum(ih, 0.0) * jnp.maximum(iw, 0.0)
    area_c = (sy2 - sy1) * (sx2 - sx1)
    area_r = (y2r - y1r) * (x2r - x1r)
    iou = inter / (area_c + area_r - inter + 1e-9)
    SUPB = iou > _NMS_T

    UT128 = _iota((128, 128), 0) < _iota((128, 128), 1)
    K0 = s_sorted > 0.0
    K = K0
    dn = (((1,), (0,)), ((), ()))
    for b in range(8):
        lo, hi = 128 * b, 128 * (b + 1)
        Dbf = (SUPB[lo:hi, lo:hi] & UT128).astype(f32)
        kb0 = K[:, lo:hi]
        kb0f = kb0.astype(f32)

        def step(kf):
            sup = jax.lax.dot_general(kf, Dbf, dn, preferred_element_type=f32)
            return kb0f * (sup < 0.5).astype(f32)

        k1 = step(kb0f)
        k2 = step(k1)

        def in_cond(c):
            kp, k, it = c
            return jnp.logical_and(it < 130, jnp.any(k != kp))

        def in_body(c):
            _, k, it = c
            ka = step(k)
            return (ka, step(ka), it + 2)

        _, kbf, _ = jax.lax.while_loop(in_cond, in_body,
                                       (k1, k2, jnp.int32(2)))
        kb = kbf > 0.5

        if b < 7:
            kbcol = jnp.transpose(kb.astype(f32), (1, 0)) > 0.5
            supL = jnp.any(SUPB[lo:hi, hi:] & kbcol, axis=0, keepdims=True)
            later = K[:, hi:] & jnp.logical_not(supL)
            if b == 0:
                K = jnp.concatenate([kb, later], axis=1)
            else:
                K = jnp.concatenate([K[:, :lo], kb, later], axis=1)
        else:
            K = jnp.concatenate([K[:, :lo], kb], axis=1)

    Kf = K.astype(f32)
    TRIUK = (ij <= ik).astype(f32)
    inclK = jax.lax.dot_general(Kf, TRIUK, (((1,), (0,)), ((), ())),
                                preferred_element_type=f32)
    pos2 = jnp.where(K, inclK - 1.0, -1.0)
    OH2 = (_iota((_OUT_SLOTS, _K), 0).astype(f32) == pos2).astype(f32)
    OUT = _onehot_matmul(OH2, S)
    out_ref[0] = OUT


def kernel(raw_cls_bbox, raw_prob):
    probs_t = raw_prob.T[1:]
    P = jnp.pad(probs_t, ((0, 0), (0, _N_PAD - _N_BOX)))
    P = P.reshape(_NCLS, _ROWS, 128)
    bb = jnp.transpose(raw_cls_bbox, (1, 2, 0))[1:]
    bb = jnp.pad(bb, ((0, 0), (0, 0), (0, _N_PAD - _N_BOX)))
    bb = bb.reshape(_NCLS, 4, _ROWS, 128)

    li = pl.pallas_call(
        _sel_body,
        out_shape=jax.ShapeDtypeStruct((_NCLS, _ROWS, 128), jnp.float32),
    )(P)

    spec = pl.BlockSpec((1, _ROWS, 128), lambda i: (i, 0, 0))
    out = pl.pallas_call(
        _nms_body,
        out_shape=jax.ShapeDtypeStruct((_NCLS, _OUT_SLOTS, 8), jnp.float32),
        grid=(_NCLS,),
        in_specs=[spec] * 6,
        out_specs=pl.BlockSpec((1, _OUT_SLOTS, 8), lambda i: (i, 0, 0)),
    )(li, P, bb[:, 0], bb[:, 1], bb[:, 2], bb[:, 3])

    return jnp.concatenate(
        [out[:, :_MAX_DET, 1:5], out[:, :_MAX_DET, 0:1]], axis=-1)
